# Initial kernel scaffold; baseline (speedup 1.0000x reference)
import jax
import jax.numpy as jnp
from jax.experimental import pallas as pl


def kernel(x, codebook):
    raise NotImplementedError("write your pallas kernel here")


# speedup vs baseline: 2.5434x; 2.5434x over previous
import jax
import jax.numpy as jnp
from jax import lax
from jax.experimental import pallas as pl

V = 1024
C = 64


def _vq_kernel(x_ref, cb_ref, xd_ref, counts_ref, perp_ref):
    i = pl.program_id(0)
    n_steps = pl.num_programs(0)

    xb = x_ref[0]
    cb = cb_ref[...]

    xsq = jnp.sum(xb * xb, axis=0, keepdims=True)
    csq = jnp.sum(cb * cb, axis=1, keepdims=True)
    mm = jnp.dot(cb, xb)
    distance = (xsq - 2.0 * mm) + csq

    minval = jnp.min(distance, axis=0, keepdims=True)
    iota_v = lax.broadcasted_iota(jnp.int32, distance.shape, 0)
    idx = jnp.min(jnp.where(distance <= minval, iota_v, V),
                  axis=0, keepdims=True)
    onehot = (iota_v == idx).astype(jnp.float32)

    xd_ref[0] = lax.dot_general(cb, onehot, (((0,), (0,)), ((), ())))

    part = jnp.sum(onehot, axis=1, keepdims=True)

    @pl.when(i == 0)
    def _():
        counts_ref[...] = part

    @pl.when(i > 0)
    def _():
        counts_ref[...] += part

    @pl.when(i == n_steps - 1)
    def _():
        counts = counts_ref[...]
        prob = counts / jnp.sum(counts)
        ent = jnp.sum(prob * jnp.log(prob + 1e-07),
                      axis=0, keepdims=True)
        perp_ref[...] = jnp.exp(-ent)


def kernel(x, codebook):
    N, width, T = x.shape
    xd, counts, perp = pl.pallas_call(
        _vq_kernel,
        grid=(N,),
        in_specs=[
            pl.BlockSpec((1, width, T), lambda i: (i, 0, 0)),
            pl.BlockSpec((V, C), lambda i: (0, 0)),
        ],
        out_specs=[
            pl.BlockSpec((1, width, T), lambda i: (i, 0, 0)),
            pl.BlockSpec((V, 1), lambda i: (0, 0)),
            pl.BlockSpec((1, 1), lambda i: (0, 0)),
        ],
        out_shape=[
            jax.ShapeDtypeStruct((N, width, T), jnp.float32),
            jax.ShapeDtypeStruct((V, 1), jnp.float32),
            jax.ShapeDtypeStruct((1, 1), jnp.float32),
        ],
    )(x, codebook)
    return (xd, perp[0, 0])
